# baseline (speedup 1.0000x reference)
import functools

import jax
import jax.numpy as jnp
from jax import lax
from jax.experimental import pallas as pl
from jax.experimental.pallas import tpu as pltpu
from jax.experimental.pallas import tpu_sc as plsc

NUM_CLASSES = 8192
CODE_DIM = 64
K = 4
CW = CODE_DIM // 2
KDW = K * CW
CHUNK = 128
GROUP = 16


def _pack_pairs(x, n_rows):
    xb = x.astype(jnp.bfloat16).reshape(n_rows, 2, 2, 16)
    xw = jnp.stack([xb[:, :, 0, :], xb[:, :, 1, :]], axis=-1)
    return lax.bitcast_convert_type(xw, jnp.int32)


def kernel(codes, pred_class, centroids):
    B = codes.shape[0]
    NC, NS = 2, 16
    NW = NC * NS
    rows_per_w = B // NW
    n_chunks = rows_per_w // CHUNK
    assert rows_per_w * NW == B and n_chunks * CHUNK == rows_per_w
    assert n_chunks % 2 == 0

    table = _pack_pairs(
        centroids.reshape(NUM_CLASSES * K, CODE_DIM), NUM_CLASSES * K
    ).reshape(NUM_CLASSES, KDW)
    mesh = plsc.VectorSubcoreMesh(core_axis_name="c", subcore_axis_name="s")

    @functools.partial(
        pl.kernel,
        mesh=mesh,
        out_type=jax.ShapeDtypeStruct((B,), jnp.float32),
        compiler_params=pltpu.CompilerParams(needs_layout_passes=False),
        scratch_types=[
            pltpu.VMEM((CHUNK,), jnp.int32),
            pltpu.VMEM((CHUNK,), jnp.int32),
            pltpu.VMEM((CHUNK, KDW), jnp.int32),
            pltpu.VMEM((CHUNK, KDW), jnp.int32),
            pltpu.VMEM((CHUNK, CODE_DIM), jnp.float32),
            pltpu.VMEM((CHUNK, CODE_DIM), jnp.float32),
            pltpu.VMEM((CHUNK,), jnp.float32),
            pltpu.SemaphoreType.DMA,
            pltpu.SemaphoreType.DMA,
            pltpu.SemaphoreType.DMA,
            pltpu.SemaphoreType.DMA,
        ],
    )
    def sc_kernel(codes_hbm, idx_hbm, table_hbm, out_hbm,
                  idx0, idx1, rows0, rows1, cod0, cod1, out_v,
                  sg0, sg1, sc0, sc1):
        idx_v = (idx0, idx1)
        rows_v = (rows0, rows1)
        codes_v = (cod0, cod1)
        sem_g = (sg0, sg1)
        sem_c = (sc0, sc1)

        wid = lax.axis_index("s") * NC + lax.axis_index("c")
        lane = lax.iota(jnp.int32, 16)
        lane_eq = [lane == j for j in range(GROUP)]

        def stage(b, ci):
            base = wid * rows_per_w + ci * CHUNK
            pltpu.sync_copy(idx_hbm.at[pl.ds(base, CHUNK)], idx_v[b])
            pltpu.async_copy(table_hbm.at[idx_v[b]], rows_v[b], sem_g[b])
            pltpu.async_copy(codes_hbm.at[pl.ds(base, CHUNK)],
                             codes_v[b], sem_c[b])

        def wait(b):
            pltpu.make_async_copy(
                table_hbm.at[idx_v[b]], rows_v[b], sem_g[b]).wait()
            pltpu.make_async_copy(
                codes_hbm.at[pl.ds(0, CHUNK)], codes_v[b], sem_c[b]).wait()

        def compute(b, ci):
            base = wid * rows_per_w + ci * CHUNK

            def group_body(g, c2):
                res = jnp.zeros((16,), jnp.float32)
                for r16 in range(GROUP):
                    r = g * GROUP + r16
                    c = [codes_v[b][r, pl.ds(16 * j, 16)]
                         for j in range(4)]
                    cbf = [plsc.pack(c[2 * h], c[2 * h + 1],
                                     format=plsc.PackFormat.INTERLEAVED)
                           for h in range(2)]
                    best = None
                    for k in range(K):
                        s = jnp.zeros((16,), jnp.float32)
                        for h in range(2):
                            t32 = rows_v[b][r, pl.ds(k * 32 + h * 16, 16)]
                            tbf = plsc.bitcast(t32, jnp.bfloat16)
                            diff = jnp.abs(cbf[h] - tbf)
                            lo, hi = plsc.unpack(
                                diff, format=plsc.PackFormat.INTERLEAVED)
                            s = s + lo + hi
                        tot = jnp.sum(s)
                        best = tot if best is None else jnp.minimum(best, tot)
                    res = jnp.where(lane_eq[r16],
                                    jnp.full((16,), best * (1.0 / CODE_DIM)),
                                    res)
                out_v[pl.ds(g * GROUP, GROUP)] = res
                return c2

            lax.fori_loop(0, CHUNK // GROUP, group_body, 0)
            pltpu.sync_copy(out_v, out_hbm.at[pl.ds(base, CHUNK)])

        stage(0, 0)

        def outer(cc, carry):
            for b in range(2):
                ci = cc * 2 + b

                @pl.when(ci + 1 < n_chunks)
                def _():
                    stage(1 - b, ci + 1)

                wait(b)
                compute(b, ci)
            return carry

        lax.fori_loop(0, n_chunks // 2, outer, 0)

    return sc_kernel(codes, pred_class, table)


# speedup vs baseline: 3.9399x; 3.9399x over previous
import functools

import jax
import jax.numpy as jnp
from jax import lax
from jax.experimental import pallas as pl
from jax.experimental.pallas import tpu as pltpu
from jax.experimental.pallas import tpu_sc as plsc

NUM_CLASSES = 8192
CODE_DIM = 64
K = 4
CW = CODE_DIM // 2
KDW = K * CW
CHUNK = 128
GROUP = 16


def _pack_table(centroids):
    tb = centroids.astype(jnp.bfloat16).reshape(NUM_CLASSES, K, 2, 2, 16)
    tw = jnp.stack([tb[:, :, :, 0, :], tb[:, :, :, 1, :]], axis=-1)
    return lax.bitcast_convert_type(tw, jnp.int32).reshape(NUM_CLASSES, KDW)


def kernel(codes, pred_class, centroids):
    B = codes.shape[0]
    NC, NS = 2, 16
    NW = NC * NS
    rows_per_w = B // NW
    n_chunks = rows_per_w // CHUNK
    assert rows_per_w * NW == B and n_chunks * CHUNK == rows_per_w
    assert n_chunks % 2 == 0

    table = _pack_table(centroids)
    mesh = plsc.VectorSubcoreMesh(core_axis_name="c", subcore_axis_name="s")

    @functools.partial(
        pl.kernel,
        mesh=mesh,
        out_type=jax.ShapeDtypeStruct((B,), jnp.float32),
        compiler_params=pltpu.CompilerParams(needs_layout_passes=False),
        scratch_types=[
            pltpu.VMEM((CHUNK,), jnp.int32),
            pltpu.VMEM((CHUNK,), jnp.int32),
            pltpu.VMEM((CHUNK, KDW), jnp.int32),
            pltpu.VMEM((CHUNK, KDW), jnp.int32),
            pltpu.VMEM((CHUNK, CODE_DIM), jnp.float32),
            pltpu.VMEM((CHUNK, CODE_DIM), jnp.float32),
            pltpu.VMEM((CHUNK,), jnp.float32),
            pltpu.SemaphoreType.DMA,
            pltpu.SemaphoreType.DMA,
            pltpu.SemaphoreType.DMA,
            pltpu.SemaphoreType.DMA,
        ],
    )
    def sc_kernel(codes_hbm, idx_hbm, table_hbm, out_hbm,
                  idx0, idx1, rows0, rows1, cod0, cod1, out_v,
                  sg0, sg1, sc0, sc1):
        idx_v = (idx0, idx1)
        rows_v = (rows0, rows1)
        codes_v = (cod0, cod1)
        sem_g = (sg0, sg1)
        sem_c = (sc0, sc1)

        wid = lax.axis_index("s") * NC + lax.axis_index("c")
        lane = lax.iota(jnp.int32, 16)
        lane_eq = [lane == j for j in range(GROUP)]

        def stage(b, ci):
            base = wid * rows_per_w + ci * CHUNK
            pltpu.sync_copy(idx_hbm.at[pl.ds(base, CHUNK)], idx_v[b])
            pltpu.async_copy(table_hbm.at[idx_v[b]], rows_v[b], sem_g[b])
            pltpu.async_copy(codes_hbm.at[pl.ds(base, CHUNK)],
                             codes_v[b], sem_c[b])

        def wait(b):
            pltpu.make_async_copy(
                table_hbm.at[idx_v[b]], rows_v[b], sem_g[b]).wait()
            pltpu.make_async_copy(
                codes_hbm.at[pl.ds(0, CHUNK)], codes_v[b], sem_c[b]).wait()

        def compute(b, ci):
            base = wid * rows_per_w + ci * CHUNK

            def group_body(g, c2):
                res = jnp.zeros((16,), jnp.float32)
                for r16 in range(GROUP):
                    r = g * GROUP + r16
                    c = [codes_v[b][r, pl.ds(16 * j, 16)]
                         for j in range(4)]
                    cbf = [plsc.pack(c[2 * h], c[2 * h + 1],
                                     format=plsc.PackFormat.INTERLEAVED)
                           for h in range(2)]
                    best = None
                    for k in range(K):
                        s = jnp.zeros((16,), jnp.float32)
                        for h in range(2):
                            t32 = rows_v[b][r, pl.ds(k * 32 + h * 16, 16)]
                            tbf = plsc.bitcast(t32, jnp.bfloat16)
                            diff = jnp.abs(cbf[h] - tbf)
                            lo, hi = plsc.unpack(
                                diff, format=plsc.PackFormat.INTERLEAVED)
                            s = s + lo + hi
                        tot = jnp.sum(s)
                        best = tot if best is None else jnp.minimum(best, tot)
                    res = jnp.where(lane_eq[r16],
                                    jnp.full((16,), best * (1.0 / CODE_DIM)),
                                    res)
                out_v[pl.ds(g * GROUP, GROUP)] = res
                return c2

            lax.fori_loop(0, CHUNK // GROUP, group_body, 0)
            pltpu.sync_copy(out_v, out_hbm.at[pl.ds(base, CHUNK)])

        stage(0, 0)

        def outer(cc, carry):
            for b in range(2):
                ci = cc * 2 + b

                @pl.when(ci + 1 < n_chunks)
                def _():
                    stage(1 - b, ci + 1)

                wait(b)
                compute(b, ci)
            return carry

        lax.fori_loop(0, n_chunks // 2, outer, 0)

    return sc_kernel(codes, pred_class, table)
